# baseline (speedup 1.0000x reference)
import functools

import jax
import jax.numpy as jnp
from jax import lax
from jax.experimental import pallas as pl
from jax.experimental.pallas import tpu as pltpu
from jax.experimental.pallas import tpu_sc as plsc

N = 10000
E = 320000
D = 128

NC = 2
NS = 16
HD = D // NC

CHUNK = 128
NCH = 160
TOT_CHUNKS = NS * NCH
E_PAD = TOT_CHUNKS * CHUNK

ROWS_PER_TILE = 640
N_PAD = NS * ROWS_PER_TILE

NB = 5

DEG_W = 16

_mesh = plsc.VectorSubcoreMesh(core_axis_name="c", subcore_axis_name="s")


def _sc_agg_body(h_hbm, src_hbm, dst_hbm, acc_hbm, idx_s, idx_d, rows,
                 acc_sh, *sems):
    c = lax.axis_index("c")
    s = lax.axis_index("s")

    z16 = jnp.zeros((16,), jnp.float32)

    def _zrow(r, _):
        for j in range(HD // 16):
            rows[0, r, pl.ds(j * 16, 16)] = z16
        return 0
    lax.fori_loop(0, CHUNK, _zrow, 0)
    for j in range(ROWS_PER_TILE // CHUNK):
        pltpu.sync_copy(rows.at[0],
                        acc_sh.at[pl.ds(s * ROWS_PER_TILE + j * CHUNK, CHUNK)])

    plsc.subcore_barrier()

    pltpu.sync_copy(src_hbm.at[pl.ds(s * NCH, NCH)], idx_s)
    pltpu.sync_copy(dst_hbm.at[pl.ds(s * NCH, NCH)], idx_d)

    htab = h_hbm.at[c]
    gsem = sems[:NB]
    ssem = sems[NB:]

    for b in range(NB):
        pltpu.async_copy(htab.at[idx_s.at[b]], rows.at[b], gsem[b])

    def _group(g, _):
        for b in range(NB):
            k = g * NB + b
            kp = k - 1
            bp = (b - 1) % NB

            @pl.when(kp >= 0)
            def _():
                pltpu.make_async_copy(htab.at[idx_s.at[kp]], rows.at[bp],
                                      ssem[bp]).wait()

                @pl.when(kp + NB < NCH)
                def _():
                    pltpu.async_copy(htab.at[idx_s.at[kp + NB]], rows.at[bp],
                                     gsem[bp])

            pltpu.make_async_copy(htab.at[idx_s.at[k]], rows.at[b],
                                  gsem[b]).wait()
            pltpu.async_copy(rows.at[b], acc_sh.at[idx_d.at[k]], ssem[b],
                             add=True)
        return 0
    lax.fori_loop(0, NCH // NB, _group, 0)

    bl = (NCH - 1) % NB
    pltpu.make_async_copy(htab.at[idx_s.at[NCH - 1]], rows.at[bl],
                          ssem[bl]).wait()

    plsc.subcore_barrier()

    r0 = s * ROWS_PER_TILE
    pltpu.sync_copy(acc_sh.at[pl.ds(r0, ROWS_PER_TILE)],
                    acc_hbm.at[c, pl.ds(r0, ROWS_PER_TILE)])


_sc_agg = pl.kernel(
    _sc_agg_body,
    out_type=(jax.ShapeDtypeStruct((NC, N_PAD, HD), jnp.float32),),
    mesh=_mesh,
    scratch_types=[
        pltpu.VMEM((NCH, CHUNK), jnp.int32),
        pltpu.VMEM((NCH, CHUNK), jnp.int32),
        pltpu.VMEM((NB, CHUNK, HD), jnp.float32),
        pltpu.MemorySpace.VMEM_SHARED((N_PAD, HD), jnp.float32),
    ] + [pltpu.SemaphoreType.DMA] * (2 * NB),
    compiler_params=pltpu.CompilerParams(use_tc_tiling_on_sc=False),
    name="sage_sc_agg")


def _sc_deg_body(dst_hbm, degp_hbm, idx_d, degbuf, ones, degacc_sh, sem):
    c = lax.axis_index("c")
    s = lax.axis_index("s")
    wid = c * NS + s

    z16 = jnp.zeros((16,), jnp.float32)
    one16 = jnp.ones((16,), jnp.float32)

    def _zdeg(r, _):
        degbuf[r, pl.ds(0, 16)] = z16
        return 0
    lax.fori_loop(0, ROWS_PER_TILE, _zdeg, 0)
    pltpu.sync_copy(degbuf, degacc_sh.at[pl.ds(s * ROWS_PER_TILE, ROWS_PER_TILE)])

    def _fones(r, _):
        ones[r, pl.ds(0, 16)] = one16
        return 0
    lax.fori_loop(0, CHUNK, _fones, 0)

    plsc.subcore_barrier()

    half = NCH // NC
    pltpu.sync_copy(dst_hbm.at[pl.ds(s * NCH + c * half, half)], idx_d)

    def _chunk(k, _):
        pltpu.sync_copy(ones, degacc_sh.at[idx_d.at[k]], add=True)
        return 0
    lax.fori_loop(0, half, _chunk, 0)

    plsc.subcore_barrier()

    r0 = s * ROWS_PER_TILE
    pltpu.sync_copy(degacc_sh.at[pl.ds(r0, ROWS_PER_TILE)], degbuf)
    pltpu.sync_copy(degbuf, degp_hbm.at[c, pl.ds(r0, ROWS_PER_TILE)])


_sc_deg = pl.kernel(
    _sc_deg_body,
    out_type=(jax.ShapeDtypeStruct((NC, N_PAD, DEG_W), jnp.float32),),
    mesh=_mesh,
    scratch_types=[
        pltpu.VMEM((NCH // NC, CHUNK), jnp.int32),
        pltpu.VMEM((ROWS_PER_TILE, DEG_W), jnp.float32),
        pltpu.VMEM((CHUNK, DEG_W), jnp.float32),
        pltpu.MemorySpace.VMEM_SHARED((N_PAD, DEG_W), jnp.float32),
        pltpu.SemaphoreType.DMA,
    ],
    compiler_params=pltpu.CompilerParams(use_tc_tiling_on_sc=False),
    name="sage_sc_deg")


_TC_R = 2048


def _tc_self_body(h_ref, ws_ref, b_ref, o_ref):
    h = jnp.concatenate([h_ref[0], h_ref[1]], axis=1)
    o_ref[...] = jnp.dot(h, ws_ref[...],
                         preferred_element_type=jnp.float32) + b_ref[...]


def _tc_self(h, ws, b):
    return pl.pallas_call(
        _tc_self_body,
        grid=(N_PAD // _TC_R,),
        in_specs=[
            pl.BlockSpec((NC, _TC_R, HD), lambda i: (0, i, 0)),
            pl.BlockSpec((D, D), lambda i: (0, 0)),
            pl.BlockSpec((1, D), lambda i: (0, 0)),
        ],
        out_specs=pl.BlockSpec((_TC_R, D), lambda i: (i, 0)),
        out_shape=jax.ShapeDtypeStruct((N_PAD, D), jnp.float32),
        name="sage_tc_self",
    )(h, ws, b)


def _tc_comb_body(hs_ref, a_ref, dp_ref, wn_ref, o_ref, *, relu, split_out):
    deg = jnp.maximum(dp_ref[0, :, 0] + dp_ref[1, :, 0], 1.0)
    agg = jnp.concatenate([a_ref[0], a_ref[1]], axis=1) / deg[:, None]
    o = hs_ref[...] + jnp.dot(agg, wn_ref[...],
                              preferred_element_type=jnp.float32)
    if relu:
        o = jnp.maximum(o, 0.0)
    if split_out:
        o_ref[0] = o[:, :HD]
        o_ref[1] = o[:, HD:]
    else:
        o_ref[...] = o


def _tc_combine(hs, acc, degp, wn, relu, split_out=True):
    if split_out:
        out_spec = pl.BlockSpec((NC, _TC_R, HD), lambda i: (0, i, 0))
        out_shape = jax.ShapeDtypeStruct((NC, N_PAD, HD), jnp.float32)
    else:
        out_spec = pl.BlockSpec((_TC_R, D), lambda i: (i, 0))
        out_shape = jax.ShapeDtypeStruct((N_PAD, D), jnp.float32)
    return pl.pallas_call(
        functools.partial(_tc_comb_body, relu=relu, split_out=split_out),
        grid=(N_PAD // _TC_R,),
        in_specs=[
            pl.BlockSpec((_TC_R, D), lambda i: (i, 0)),
            pl.BlockSpec((NC, _TC_R, HD), lambda i: (0, i, 0)),
            pl.BlockSpec((NC, _TC_R, DEG_W), lambda i: (0, i, 0)),
            pl.BlockSpec((D, D), lambda i: (0, 0)),
        ],
        out_specs=out_spec,
        out_shape=out_shape,
        name="sage_tc_combine",
    )(hs, acc, degp, wn)


def kernel(features, edge_index, W_self0, W_neigh0, b0, W_self1, W_neigh1,
           b1, W_self2, W_neigh2, b2):
    fpad = jnp.concatenate(
        [features, jnp.zeros((N_PAD - N, D), jnp.float32)], axis=0)
    h0 = jnp.stack([fpad[:, :HD], fpad[:, HD:]])

    ei = edge_index.astype(jnp.int32)
    pad = N + (jnp.arange(E_PAD - E, dtype=jnp.int32) % (N_PAD - N))
    src = jnp.concatenate([ei[0], pad]).reshape(TOT_CHUNKS, CHUNK)
    dst = jnp.concatenate([ei[1], pad]).reshape(TOT_CHUNKS, CHUNK)

    (degp,) = _sc_deg(dst)
    (acc,) = _sc_agg(h0, src, dst)
    hs = _tc_self(h0, W_self0, b0.reshape(1, D))
    h1 = _tc_combine(hs, acc, degp, W_neigh0, True)
    (acc,) = _sc_agg(h1, src, dst)
    hs = _tc_self(h1, W_self1, b1.reshape(1, D))
    h2 = _tc_combine(hs, acc, degp, W_neigh1, True)
    (acc,) = _sc_agg(h2, src, dst)
    hs = _tc_self(h2, W_self2, b2.reshape(1, D))
    h3 = _tc_combine(hs, acc, degp, W_neigh2, False, split_out=False)
    return h3[:N]


# speedup vs baseline: 1.2990x; 1.2990x over previous
import functools

import jax
import jax.numpy as jnp
from jax import lax
from jax.experimental import pallas as pl
from jax.experimental.pallas import tpu as pltpu
from jax.experimental.pallas import tpu_sc as plsc

N = 10000
E = 320000
D = 128

NC = 2
NS = 16
HD = D // NC

CHUNK = 128
NCH = 160
TOT_CHUNKS = NS * NCH
E_PAD = TOT_CHUNKS * CHUNK

ROWS_PER_TILE = 640
N_PAD = NS * ROWS_PER_TILE

NB = 5

DEG_W = 16

_mesh = plsc.VectorSubcoreMesh(core_axis_name="c", subcore_axis_name="s")


def _sc_agg_body(h_hbm, src_hbm, dst_hbm, acc_hbm, idx_s, idx_d, rows,
                 acc_sh, *sems):
    c = lax.axis_index("c")
    s = lax.axis_index("s")

    z32 = jnp.zeros((32,), jnp.bfloat16)

    def _zrow(r, _):
        for j in range(HD // 32):
            rows[0, r, pl.ds(j * 32, 32)] = z32
        return 0
    lax.fori_loop(0, CHUNK, _zrow, 0)
    for j in range(ROWS_PER_TILE // CHUNK):
        pltpu.sync_copy(rows.at[0],
                        acc_sh.at[pl.ds(s * ROWS_PER_TILE + j * CHUNK, CHUNK)])

    plsc.subcore_barrier()

    pltpu.sync_copy(src_hbm.at[pl.ds(s * NCH, NCH)], idx_s)
    pltpu.sync_copy(dst_hbm.at[pl.ds(s * NCH, NCH)], idx_d)

    htab = h_hbm.at[c]
    gsem = sems[:NB]
    ssem = sems[NB:]

    for b in range(NB):
        pltpu.async_copy(htab.at[idx_s.at[b]], rows.at[b], gsem[b])

    def _group(g, _):
        for b in range(NB):
            k = g * NB + b
            kp = k - 1
            bp = (b - 1) % NB

            @pl.when(kp >= 0)
            def _():
                pltpu.make_async_copy(htab.at[idx_s.at[kp]], rows.at[bp],
                                      ssem[bp]).wait()

                @pl.when(kp + NB < NCH)
                def _():
                    pltpu.async_copy(htab.at[idx_s.at[kp + NB]], rows.at[bp],
                                     gsem[bp])

            pltpu.make_async_copy(htab.at[idx_s.at[k]], rows.at[b],
                                  gsem[b]).wait()
            pltpu.async_copy(rows.at[b], acc_sh.at[idx_d.at[k]], ssem[b],
                             add=True)
        return 0
    lax.fori_loop(0, NCH // NB, _group, 0)

    bl = (NCH - 1) % NB
    pltpu.make_async_copy(htab.at[idx_s.at[NCH - 1]], rows.at[bl],
                          ssem[bl]).wait()

    plsc.subcore_barrier()

    r0 = s * ROWS_PER_TILE
    pltpu.sync_copy(acc_sh.at[pl.ds(r0, ROWS_PER_TILE)],
                    acc_hbm.at[c, pl.ds(r0, ROWS_PER_TILE)])


_sc_agg = pl.kernel(
    _sc_agg_body,
    out_type=(jax.ShapeDtypeStruct((NC, N_PAD, HD), jnp.bfloat16),),
    mesh=_mesh,
    scratch_types=[
        pltpu.VMEM((NCH, CHUNK), jnp.int32),
        pltpu.VMEM((NCH, CHUNK), jnp.int32),
        pltpu.VMEM((NB, CHUNK, HD), jnp.bfloat16),
        pltpu.MemorySpace.VMEM_SHARED((N_PAD, HD), jnp.bfloat16),
    ] + [pltpu.SemaphoreType.DMA] * (2 * NB),
    compiler_params=pltpu.CompilerParams(use_tc_tiling_on_sc=False),
    name="sage_sc_agg")


def _sc_deg_body(dst_hbm, degp_hbm, idx_d, degbuf, ones, degacc_sh, sem):
    c = lax.axis_index("c")
    s = lax.axis_index("s")
    wid = c * NS + s

    z16 = jnp.zeros((16,), jnp.float32)
    one16 = jnp.ones((16,), jnp.float32)

    def _zdeg(r, _):
        degbuf[r, pl.ds(0, 16)] = z16
        return 0
    lax.fori_loop(0, ROWS_PER_TILE, _zdeg, 0)
    pltpu.sync_copy(degbuf, degacc_sh.at[pl.ds(s * ROWS_PER_TILE, ROWS_PER_TILE)])

    def _fones(r, _):
        ones[r, pl.ds(0, 16)] = one16
        return 0
    lax.fori_loop(0, CHUNK, _fones, 0)

    plsc.subcore_barrier()

    half = NCH // NC
    pltpu.sync_copy(dst_hbm.at[pl.ds(s * NCH + c * half, half)], idx_d)

    def _chunk(k, _):
        pltpu.sync_copy(ones, degacc_sh.at[idx_d.at[k]], add=True)
        return 0
    lax.fori_loop(0, half, _chunk, 0)

    plsc.subcore_barrier()

    r0 = s * ROWS_PER_TILE
    pltpu.sync_copy(degacc_sh.at[pl.ds(r0, ROWS_PER_TILE)], degbuf)
    pltpu.sync_copy(degbuf, degp_hbm.at[c, pl.ds(r0, ROWS_PER_TILE)])


_sc_deg = pl.kernel(
    _sc_deg_body,
    out_type=(jax.ShapeDtypeStruct((NC, N_PAD, DEG_W), jnp.float32),),
    mesh=_mesh,
    scratch_types=[
        pltpu.VMEM((NCH // NC, CHUNK), jnp.int32),
        pltpu.VMEM((ROWS_PER_TILE, DEG_W), jnp.float32),
        pltpu.VMEM((CHUNK, DEG_W), jnp.float32),
        pltpu.MemorySpace.VMEM_SHARED((N_PAD, DEG_W), jnp.float32),
        pltpu.SemaphoreType.DMA,
    ],
    compiler_params=pltpu.CompilerParams(use_tc_tiling_on_sc=False),
    name="sage_sc_deg")


def _tc_body(h_ref, a_ref, dp_ref, ws_ref, wn_ref, b_ref, o_ref, *refs,
             relu, split_out):
    deg = jnp.maximum(dp_ref[0, :, 0] + dp_ref[1, :, 0], 1.0)
    h = jnp.concatenate([h_ref[0], h_ref[1]], axis=1)
    agg = jnp.concatenate([a_ref[0], a_ref[1]],
                          axis=1).astype(jnp.float32) / deg[:, None]
    o = jnp.dot(h, ws_ref[...], preferred_element_type=jnp.float32)
    o = o + jnp.dot(agg, wn_ref[...], preferred_element_type=jnp.float32)
    o = o + b_ref[...]
    if relu:
        o = jnp.maximum(o, 0.0)
    if split_out:
        ob_ref = refs[0]
        o_ref[0] = o[:, :HD]
        o_ref[1] = o[:, HD:]
        ob = o.astype(jnp.bfloat16)
        ob_ref[0] = ob[:, :HD]
        ob_ref[1] = ob[:, HD:]
    else:
        o_ref[...] = o


_TC_R = 2048


def _tc_layer(h, acc, degp, ws, wn, b, relu, split_out=True):
    grid = (N_PAD // _TC_R,)
    if split_out:
        out_spec = [pl.BlockSpec((NC, _TC_R, HD), lambda i: (0, i, 0)),
                    pl.BlockSpec((NC, _TC_R, HD), lambda i: (0, i, 0))]
        out_shape = [jax.ShapeDtypeStruct((NC, N_PAD, HD), jnp.float32),
                     jax.ShapeDtypeStruct((NC, N_PAD, HD), jnp.bfloat16)]
    else:
        out_spec = pl.BlockSpec((_TC_R, D), lambda i: (i, 0))
        out_shape = jax.ShapeDtypeStruct((N_PAD, D), jnp.float32)
    return pl.pallas_call(
        functools.partial(_tc_body, relu=relu, split_out=split_out),
        grid=grid,
        in_specs=[
            pl.BlockSpec((NC, _TC_R, HD), lambda i: (0, i, 0)),
            pl.BlockSpec((NC, _TC_R, HD), lambda i: (0, i, 0)),
            pl.BlockSpec((NC, _TC_R, DEG_W), lambda i: (0, i, 0)),
            pl.BlockSpec((D, D), lambda i: (0, 0)),
            pl.BlockSpec((D, D), lambda i: (0, 0)),
            pl.BlockSpec((1, D), lambda i: (0, 0)),
        ],
        out_specs=out_spec,
        out_shape=out_shape,
        name="sage_tc_dense",
    )(h, acc, degp, ws, wn, b)


def kernel(features, edge_index, W_self0, W_neigh0, b0, W_self1, W_neigh1,
           b1, W_self2, W_neigh2, b2):
    fpad = jnp.concatenate(
        [features, jnp.zeros((N_PAD - N, D), jnp.float32)], axis=0)
    h0 = jnp.stack([fpad[:, :HD], fpad[:, HD:]])

    ei = edge_index.astype(jnp.int32)
    pad = N + (jnp.arange(E_PAD - E, dtype=jnp.int32) % (N_PAD - N))
    src = jnp.concatenate([ei[0], pad]).reshape(TOT_CHUNKS, CHUNK)
    dst = jnp.concatenate([ei[1], pad]).reshape(TOT_CHUNKS, CHUNK)

    h0b = h0.astype(jnp.bfloat16)
    (degp,) = _sc_deg(dst)
    (acc,) = _sc_agg(h0b, src, dst)
    h1, h1b = _tc_layer(h0, acc, degp, W_self0, W_neigh0, b0.reshape(1, D),
                        True)
    (acc,) = _sc_agg(h1b, src, dst)
    h2, h2b = _tc_layer(h1, acc, degp, W_self1, W_neigh1, b1.reshape(1, D),
                        True)
    (acc,) = _sc_agg(h2b, src, dst)
    h3 = _tc_layer(h2, acc, degp, W_self2, W_neigh2, b2.reshape(1, D), False,
                   split_out=False)
    return h3[:N]


# speedup vs baseline: 1.3057x; 1.0051x over previous
import functools

import jax
import jax.numpy as jnp
from jax import lax
from jax.experimental import pallas as pl
from jax.experimental.pallas import tpu as pltpu
from jax.experimental.pallas import tpu_sc as plsc

N = 10000
E = 320000
D = 128

NC = 2
NS = 16
HD = D // NC

CHUNK = 128
NCH = 160
TOT_CHUNKS = NS * NCH
E_PAD = TOT_CHUNKS * CHUNK

ROWS_PER_TILE = 640
N_PAD = NS * ROWS_PER_TILE

NB = 8

DEG_W = 16

_mesh = plsc.VectorSubcoreMesh(core_axis_name="c", subcore_axis_name="s")


def _sc_agg_body(h_hbm, src_hbm, dst_hbm, acc_hbm, idx_s, idx_d, rows,
                 acc_sh, *sems):
    c = lax.axis_index("c")
    s = lax.axis_index("s")

    z32 = jnp.zeros((32,), jnp.bfloat16)

    def _zrow(r, _):
        for j in range(HD // 32):
            rows[0, r, pl.ds(j * 32, 32)] = z32
        return 0
    lax.fori_loop(0, CHUNK, _zrow, 0)
    for j in range(ROWS_PER_TILE // CHUNK):
        pltpu.sync_copy(rows.at[0],
                        acc_sh.at[pl.ds(s * ROWS_PER_TILE + j * CHUNK, CHUNK)])

    plsc.subcore_barrier()

    pltpu.sync_copy(src_hbm.at[pl.ds(s * NCH, NCH)], idx_s)
    pltpu.sync_copy(dst_hbm.at[pl.ds(s * NCH, NCH)], idx_d)

    htab = h_hbm.at[c]
    gsem = sems[:NB]
    ssem = sems[NB:]

    for b in range(NB):
        pltpu.async_copy(htab.at[idx_s.at[b]], rows.at[b], gsem[b])

    def _group(g, _):
        for b in range(NB):
            k = g * NB + b
            kp = k - 1
            bp = (b - 1) % NB

            @pl.when(kp >= 0)
            def _():
                pltpu.make_async_copy(htab.at[idx_s.at[kp]], rows.at[bp],
                                      ssem[bp]).wait()

                @pl.when(kp + NB < NCH)
                def _():
                    pltpu.async_copy(htab.at[idx_s.at[kp + NB]], rows.at[bp],
                                     gsem[bp])

            pltpu.make_async_copy(htab.at[idx_s.at[k]], rows.at[b],
                                  gsem[b]).wait()
            pltpu.async_copy(rows.at[b], acc_sh.at[idx_d.at[k]], ssem[b],
                             add=True)
        return 0
    lax.fori_loop(0, NCH // NB, _group, 0)

    bl = (NCH - 1) % NB
    pltpu.make_async_copy(htab.at[idx_s.at[NCH - 1]], rows.at[bl],
                          ssem[bl]).wait()

    plsc.subcore_barrier()

    r0 = s * ROWS_PER_TILE
    pltpu.sync_copy(acc_sh.at[pl.ds(r0, ROWS_PER_TILE)],
                    acc_hbm.at[c, pl.ds(r0, ROWS_PER_TILE)])


_sc_agg = pl.kernel(
    _sc_agg_body,
    out_type=(jax.ShapeDtypeStruct((NC, N_PAD, HD), jnp.bfloat16),),
    mesh=_mesh,
    scratch_types=[
        pltpu.VMEM((NCH, CHUNK), jnp.int32),
        pltpu.VMEM((NCH, CHUNK), jnp.int32),
        pltpu.VMEM((NB, CHUNK, HD), jnp.bfloat16),
        pltpu.MemorySpace.VMEM_SHARED((N_PAD, HD), jnp.bfloat16),
    ] + [pltpu.SemaphoreType.DMA] * (2 * NB),
    compiler_params=pltpu.CompilerParams(use_tc_tiling_on_sc=False),
    name="sage_sc_agg")


def _sc_deg_body(dst_hbm, degp_hbm, idx_d, degbuf, ones, degacc_sh, sem):
    c = lax.axis_index("c")
    s = lax.axis_index("s")
    wid = c * NS + s

    z16 = jnp.zeros((16,), jnp.float32)
    one16 = jnp.ones((16,), jnp.float32)

    def _zdeg(r, _):
        degbuf[r, pl.ds(0, 16)] = z16
        return 0
    lax.fori_loop(0, ROWS_PER_TILE, _zdeg, 0)
    pltpu.sync_copy(degbuf, degacc_sh.at[pl.ds(s * ROWS_PER_TILE, ROWS_PER_TILE)])

    def _fones(r, _):
        ones[r, pl.ds(0, 16)] = one16
        return 0
    lax.fori_loop(0, CHUNK, _fones, 0)

    plsc.subcore_barrier()

    half = NCH // NC
    pltpu.sync_copy(dst_hbm.at[pl.ds(s * NCH + c * half, half)], idx_d)

    def _chunk(k, _):
        pltpu.sync_copy(ones, degacc_sh.at[idx_d.at[k]], add=True)
        return 0
    lax.fori_loop(0, half, _chunk, 0)

    plsc.subcore_barrier()

    r0 = s * ROWS_PER_TILE
    pltpu.sync_copy(degacc_sh.at[pl.ds(r0, ROWS_PER_TILE)], degbuf)
    pltpu.sync_copy(degbuf, degp_hbm.at[c, pl.ds(r0, ROWS_PER_TILE)])


_sc_deg = pl.kernel(
    _sc_deg_body,
    out_type=(jax.ShapeDtypeStruct((NC, N_PAD, DEG_W), jnp.float32),),
    mesh=_mesh,
    scratch_types=[
        pltpu.VMEM((NCH // NC, CHUNK), jnp.int32),
        pltpu.VMEM((ROWS_PER_TILE, DEG_W), jnp.float32),
        pltpu.VMEM((CHUNK, DEG_W), jnp.float32),
        pltpu.MemorySpace.VMEM_SHARED((N_PAD, DEG_W), jnp.float32),
        pltpu.SemaphoreType.DMA,
    ],
    compiler_params=pltpu.CompilerParams(use_tc_tiling_on_sc=False),
    name="sage_sc_deg")


def _tc_body(h_ref, a_ref, dp_ref, ws_ref, wn_ref, b_ref, o_ref, *refs,
             relu, split_out):
    deg = jnp.maximum(dp_ref[0, :, 0] + dp_ref[1, :, 0], 1.0)
    h = jnp.concatenate([h_ref[0], h_ref[1]], axis=1)
    agg = jnp.concatenate([a_ref[0], a_ref[1]],
                          axis=1).astype(jnp.float32) / deg[:, None]
    o = jnp.dot(h, ws_ref[...], preferred_element_type=jnp.float32)
    o = o + jnp.dot(agg, wn_ref[...], preferred_element_type=jnp.float32)
    o = o + b_ref[...]
    if relu:
        o = jnp.maximum(o, 0.0)
    if split_out:
        ob_ref = refs[0]
        o_ref[0] = o[:, :HD]
        o_ref[1] = o[:, HD:]
        ob = o.astype(jnp.bfloat16)
        ob_ref[0] = ob[:, :HD]
        ob_ref[1] = ob[:, HD:]
    else:
        o_ref[...] = o


_TC_R = 2048


def _tc_layer(h, acc, degp, ws, wn, b, relu, split_out=True):
    grid = (N_PAD // _TC_R,)
    if split_out:
        out_spec = [pl.BlockSpec((NC, _TC_R, HD), lambda i: (0, i, 0)),
                    pl.BlockSpec((NC, _TC_R, HD), lambda i: (0, i, 0))]
        out_shape = [jax.ShapeDtypeStruct((NC, N_PAD, HD), jnp.float32),
                     jax.ShapeDtypeStruct((NC, N_PAD, HD), jnp.bfloat16)]
    else:
        out_spec = pl.BlockSpec((_TC_R, D), lambda i: (i, 0))
        out_shape = jax.ShapeDtypeStruct((N_PAD, D), jnp.float32)
    return pl.pallas_call(
        functools.partial(_tc_body, relu=relu, split_out=split_out),
        grid=grid,
        in_specs=[
            pl.BlockSpec((NC, _TC_R, HD), lambda i: (0, i, 0)),
            pl.BlockSpec((NC, _TC_R, HD), lambda i: (0, i, 0)),
            pl.BlockSpec((NC, _TC_R, DEG_W), lambda i: (0, i, 0)),
            pl.BlockSpec((D, D), lambda i: (0, 0)),
            pl.BlockSpec((D, D), lambda i: (0, 0)),
            pl.BlockSpec((1, D), lambda i: (0, 0)),
        ],
        out_specs=out_spec,
        out_shape=out_shape,
        name="sage_tc_dense",
    )(h, acc, degp, ws, wn, b)


def kernel(features, edge_index, W_self0, W_neigh0, b0, W_self1, W_neigh1,
           b1, W_self2, W_neigh2, b2):
    fpad = jnp.concatenate(
        [features, jnp.zeros((N_PAD - N, D), jnp.float32)], axis=0)
    h0 = jnp.stack([fpad[:, :HD], fpad[:, HD:]])

    ei = edge_index.astype(jnp.int32)
    pad = N + (jnp.arange(E_PAD - E, dtype=jnp.int32) % (N_PAD - N))
    src = jnp.concatenate([ei[0], pad]).reshape(TOT_CHUNKS, CHUNK)
    dst = jnp.concatenate([ei[1], pad]).reshape(TOT_CHUNKS, CHUNK)

    h0b = h0.astype(jnp.bfloat16)
    (degp,) = _sc_deg(dst)
    (acc,) = _sc_agg(h0b, src, dst)
    h1, h1b = _tc_layer(h0, acc, degp, W_self0, W_neigh0, b0.reshape(1, D),
                        True)
    (acc,) = _sc_agg(h1b, src, dst)
    h2, h2b = _tc_layer(h1, acc, degp, W_self1, W_neigh1, b1.reshape(1, D),
                        True)
    (acc,) = _sc_agg(h2b, src, dst)
    h3 = _tc_layer(h2, acc, degp, W_self2, W_neigh2, b2.reshape(1, D), False,
                   split_out=False)
    return h3[:N]
